# baseline (speedup 1.0000x reference)
import functools

import jax
import jax.numpy as jnp
from jax import lax
from jax.experimental import pallas as pl
from jax.experimental.pallas import tpu as pltpu
from jax.experimental.pallas import tpu_sc as plsc

NC = 2
NS = 16
L = 16
NW = NC * NS


def _make(E, D, K, nacc=8, interpret=False):
    epw = E // NW
    nchunk = -(-epw // K)
    last_off = epw - K
    assert E % NW == 0 and K % L == 0 and D % nacc == 0
    assert epw >= K and last_off % 8 == 0 and K <= 128

    mesh = plsc.VectorSubcoreMesh(
        core_axis_name="c", subcore_axis_name="s", num_cores=NC
    )

    @functools.partial(
        pl.kernel,
        mesh=mesh,
        out_type=jax.ShapeDtypeStruct((E,), jnp.float32),
        scratch_types=[
            pltpu.VMEM((epw,), jnp.int32),
            pltpu.VMEM((epw,), jnp.int32),
            pltpu.VMEM((epw,), jnp.float32),
            pltpu.VMEM((K, D), jnp.float32),
            pltpu.VMEM((K, D), jnp.float32),
            pltpu.SemaphoreType.DMA,
        ],
        compiler_params=pltpu.CompilerParams(
            use_tc_tiling_on_sc=False, needs_layout_passes=False
        ),
        interpret=interpret,
    )
    def recon(z_hbm, si_hbm, di_hbm, out_hbm, si_v, di_v, out_v, rs_v, rd_v,
              sem):
        wid = lax.axis_index("s") * NC + lax.axis_index("c")
        base = wid * epw
        pltpu.sync_copy(si_hbm.at[pl.ds(base, epw)], si_v)
        pltpu.sync_copy(di_hbm.at[pl.ds(base, epw)], di_v)

        lane = lax.iota(jnp.int32, L)

        def chunk_body(c, carry):
            off = jnp.minimum(c * K, last_off)
            cp_s = pltpu.async_copy(z_hbm.at[si_v.at[pl.ds(off, K)]], rs_v,
                                    sem)
            cp_d = pltpu.async_copy(z_hbm.at[di_v.at[pl.ds(off, K)]], rd_v,
                                    sem)
            cp_s.wait()
            cp_d.wait()

            def group_body(g, carry2):
                eids = lane + g * L
                zero = jnp.zeros((L,), jnp.float32)

                @plsc.parallel_loop(
                    0, D, step=nacc, unroll=2, carry=(zero,) * nacc
                )
                def col_loop(d0, accs):
                    accs = list(accs)
                    dvec0 = jnp.full((L,), d0, jnp.int32)
                    for j in range(nacc):
                        dvec = dvec0 + j
                        s = plsc.load_gather(rs_v, [eids, dvec])
                        t = plsc.load_gather(rd_v, [eids, dvec])
                        accs[j] = accs[j] + s * t
                    return tuple(accs)

                accs = list(col_loop)
                while len(accs) > 1:
                    accs = [a + b for a, b in zip(accs[::2], accs[1::2])]
                dot = accs[0]
                sig = 1.0 / (1.0 + jnp.exp(-dot))
                out_v[pl.ds(off + g * L, L)] = sig
                return carry2

            lax.fori_loop(0, K // L, group_body, 0)
            return carry

        lax.fori_loop(0, nchunk, chunk_body, 0)
        pltpu.sync_copy(out_v, out_hbm.at[pl.ds(base, epw)])

    return recon


_recon = _make(160000, 256, 128)


def kernel(z, edge_index):
    ei = edge_index.astype(jnp.int32)
    return _recon(z, ei[0], ei[1])


# speedup vs baseline: 5.2695x; 5.2695x over previous
import functools

import jax
import jax.numpy as jnp
from jax import lax
from jax.experimental import pallas as pl
from jax.experimental.pallas import tpu as pltpu
from jax.experimental.pallas import tpu_sc as plsc

NC = 2
NS = 16
L = 16
NW = NC * NS


def _make(E, D, K, nacc=8, interpret=False):
    epw = E // NW
    nchunk = -(-epw // K)
    last_off = epw - K
    assert E % NW == 0 and K % L == 0 and D % nacc == 0
    assert epw >= K and last_off % 8 == 0 and K <= 128

    mesh = plsc.VectorSubcoreMesh(
        core_axis_name="c", subcore_axis_name="s", num_cores=NC
    )

    @functools.partial(
        pl.kernel,
        mesh=mesh,
        out_type=jax.ShapeDtypeStruct((E,), jnp.float32),
        scratch_types=[
            pltpu.VMEM((epw,), jnp.int32),
            pltpu.VMEM((epw,), jnp.int32),
            pltpu.VMEM((epw,), jnp.float32),
            pltpu.VMEM((K, D), jnp.float32),
            pltpu.VMEM((K, D), jnp.float32),
            pltpu.VMEM((L, L + 1), jnp.float32),
            pltpu.SemaphoreType.DMA,
        ],
        compiler_params=pltpu.CompilerParams(
            use_tc_tiling_on_sc=False, needs_layout_passes=False
        ),
        interpret=interpret,
    )
    def recon(z_hbm, si_hbm, di_hbm, out_hbm, si_v, di_v, out_v, rs_v, rd_v,
              acc_v, sem):
        wid = lax.axis_index("s") * NC + lax.axis_index("c")
        base = wid * epw
        pltpu.sync_copy(si_hbm.at[pl.ds(base, epw)], si_v)
        pltpu.sync_copy(di_hbm.at[pl.ds(base, epw)], di_v)

        lane = lax.iota(jnp.int32, L)

        def chunk_body(c, carry):
            off = jnp.minimum(c * K, last_off)
            cp_s = pltpu.async_copy(z_hbm.at[si_v.at[pl.ds(off, K)]], rs_v,
                                    sem)
            cp_d = pltpu.async_copy(z_hbm.at[di_v.at[pl.ds(off, K)]], rd_v,
                                    sem)
            cp_s.wait()
            cp_d.wait()

            def group_body(g, carry2):
                @plsc.parallel_loop(0, L, step=1, unroll=2)
                def edge_loop(i):
                    e = g * L + i
                    accs = [jnp.zeros((L,), jnp.float32) for _ in range(nacc)]
                    for j in range(D // L):
                        s = rs_v[e, pl.ds(j * L, L)]
                        t = rd_v[e, pl.ds(j * L, L)]
                        accs[j % nacc] = accs[j % nacc] + s * t
                    while len(accs) > 1:
                        accs = [a + b for a, b in zip(accs[::2], accs[1::2])]
                    acc_v[i, pl.ds(0, L)] = accs[0]

                tots = [jnp.zeros((L,), jnp.float32) for _ in range(4)]
                for c in range(L):
                    cvec = jnp.full((L,), c, jnp.int32)
                    tots[c % 4] = tots[c % 4] + plsc.load_gather(
                        acc_v, [lane, cvec])
                dot = (tots[0] + tots[1]) + (tots[2] + tots[3])
                sig = 1.0 / (1.0 + jnp.exp(-dot))
                out_v[pl.ds(off + g * L, L)] = sig
                return carry2

            lax.fori_loop(0, K // L, group_body, 0)
            return carry

        lax.fori_loop(0, nchunk, chunk_body, 0)
        pltpu.sync_copy(out_v, out_hbm.at[pl.ds(base, epw)])

    return recon


_recon = _make(160000, 256, 128)


def kernel(z, edge_index):
    ei = edge_index.astype(jnp.int32)
    return _recon(z, ei[0], ei[1])


# speedup vs baseline: 8.1781x; 1.5520x over previous
import functools

import jax
import jax.numpy as jnp
from jax import lax
from jax.experimental import pallas as pl
from jax.experimental.pallas import tpu as pltpu
from jax.experimental.pallas import tpu_sc as plsc

NC = 2
NS = 16
L = 16
NW = NC * NS


def _make(E, D, K, nacc=8):
    epw = E // NW
    nchunk = -(-epw // K)
    last_off = epw - K
    npair = -(-nchunk // 2)
    assert E % NW == 0 and K % L == 0 and D % nacc == 0
    assert epw >= K and last_off % 8 == 0 and K <= 128

    mesh = plsc.VectorSubcoreMesh(
        core_axis_name="c", subcore_axis_name="s", num_cores=NC
    )

    @functools.partial(
        pl.kernel,
        mesh=mesh,
        out_type=jax.ShapeDtypeStruct((E,), jnp.float32),
        scratch_types=[
            pltpu.VMEM((epw,), jnp.int32),
            pltpu.VMEM((epw,), jnp.int32),
            pltpu.VMEM((epw,), jnp.float32),
            pltpu.VMEM((K, D), jnp.float32),
            pltpu.VMEM((K, D), jnp.float32),
            pltpu.VMEM((K, D), jnp.float32),
            pltpu.VMEM((K, D), jnp.float32),
            pltpu.VMEM((L, L + 1), jnp.float32),
            pltpu.SemaphoreType.DMA,
            pltpu.SemaphoreType.DMA,
        ],
        compiler_params=pltpu.CompilerParams(
            use_tc_tiling_on_sc=False, needs_layout_passes=False
        ),
    )
    def recon(z_hbm, si_hbm, di_hbm, out_hbm, si_v, di_v, out_v,
              rs0, rd0, rs1, rd1, acc_v, sem0, sem1):
        wid = lax.axis_index("s") * NC + lax.axis_index("c")
        base = wid * epw
        pltpu.sync_copy(si_hbm.at[pl.ds(base, epw)], si_v)
        pltpu.sync_copy(di_hbm.at[pl.ds(base, epw)], di_v)

        lane = lax.iota(jnp.int32, L)
        bufs = ((rs0, rd0, sem0), (rs1, rd1, sem1))

        def off_of(c):
            return jnp.minimum(c * K, last_off)

        def start(c, p):
            rs, rd, sem = bufs[p]
            off = off_of(c)
            pltpu.async_copy(z_hbm.at[si_v.at[pl.ds(off, K)]], rs, sem)
            pltpu.async_copy(z_hbm.at[di_v.at[pl.ds(off, K)]], rd, sem)

        def wait(p):
            rs, rd, sem = bufs[p]
            pltpu.make_async_copy(
                z_hbm.at[si_v.at[pl.ds(0, K)]], rs, sem).wait()
            pltpu.make_async_copy(
                z_hbm.at[di_v.at[pl.ds(0, K)]], rd, sem).wait()

        def compute(c, p):
            rs, rd, _ = bufs[p]
            off = off_of(c)

            def group_body(g, carry2):
                @plsc.parallel_loop(0, L, step=1, unroll=2)
                def edge_loop(i):
                    e = g * L + i
                    accs = [jnp.zeros((L,), jnp.float32)
                            for _ in range(nacc)]
                    for j in range(D // L):
                        s = rs[e, pl.ds(j * L, L)]
                        t = rd[e, pl.ds(j * L, L)]
                        accs[j % nacc] = accs[j % nacc] + s * t
                    while len(accs) > 1:
                        accs = [a + b for a, b in zip(accs[::2], accs[1::2])]
                    acc_v[i, pl.ds(0, L)] = accs[0]

                tots = [jnp.zeros((L,), jnp.float32) for _ in range(4)]
                for col in range(L):
                    cvec = jnp.full((L,), col, jnp.int32)
                    tots[col % 4] = tots[col % 4] + plsc.load_gather(
                        acc_v, [lane, cvec])
                dot = (tots[0] + tots[1]) + (tots[2] + tots[3])
                sig = 1.0 / (1.0 + jnp.exp(-dot))
                out_v[pl.ds(off + g * L, L)] = sig
                return carry2

            lax.fori_loop(0, K // L, group_body, 0)

        start(0, 0)

        def pair_body(i, carry):
            for b in range(2):
                c = i * 2 + b

                @pl.when(c + 1 < nchunk)
                def _():
                    start(c + 1, 1 - b)

                @pl.when(c < nchunk)
                def _():
                    wait(b)
                    compute(c, b)
            return carry

        lax.fori_loop(0, npair, pair_body, 0)
        pltpu.sync_copy(out_v, out_hbm.at[pl.ds(base, epw)])

    return recon


_recon = _make(160000, 256, 64)


def kernel(z, edge_index):
    ei = edge_index.astype(jnp.int32)
    return _recon(z, ei[0], ei[1])


# speedup vs baseline: 10.1466x; 1.2407x over previous
import functools

import jax
import jax.numpy as jnp
from jax import lax
from jax.experimental import pallas as pl
from jax.experimental.pallas import tpu as pltpu
from jax.experimental.pallas import tpu_sc as plsc

NC = 2
NS = 16
L = 16
NW = NC * NS


def _make(E, D, K, nacc=8):
    epw = E // NW
    nchunk = -(-epw // K)
    last_off = epw - K
    npair = -(-nchunk // 2)
    assert E % NW == 0 and K % L == 0 and D % nacc == 0
    assert epw >= K and last_off % 8 == 0 and K <= 128

    mesh = plsc.VectorSubcoreMesh(
        core_axis_name="c", subcore_axis_name="s", num_cores=NC
    )

    @functools.partial(
        pl.kernel,
        mesh=mesh,
        out_type=jax.ShapeDtypeStruct((E,), jnp.float32),
        scratch_types=[
            pltpu.VMEM((epw,), jnp.int32),
            pltpu.VMEM((epw,), jnp.int32),
            pltpu.VMEM((epw,), jnp.float32),
            pltpu.VMEM((K, D), jnp.bfloat16),
            pltpu.VMEM((K, D), jnp.bfloat16),
            pltpu.VMEM((K, D), jnp.bfloat16),
            pltpu.VMEM((K, D), jnp.bfloat16),
            pltpu.VMEM((L, L + 1), jnp.float32),
            pltpu.SemaphoreType.DMA,
            pltpu.SemaphoreType.DMA,
        ],
        compiler_params=pltpu.CompilerParams(
            use_tc_tiling_on_sc=False, needs_layout_passes=False
        ),
    )
    def recon(z_hbm, si_hbm, di_hbm, out_hbm, si_v, di_v, out_v,
              rs0, rd0, rs1, rd1, acc_v, sem0, sem1):
        wid = lax.axis_index("s") * NC + lax.axis_index("c")
        base = wid * epw
        pltpu.sync_copy(si_hbm.at[pl.ds(base, epw)], si_v)
        pltpu.sync_copy(di_hbm.at[pl.ds(base, epw)], di_v)

        lane = lax.iota(jnp.int32, L)
        bufs = ((rs0, rd0, sem0), (rs1, rd1, sem1))

        def off_of(c):
            return jnp.minimum(c * K, last_off)

        def start(c, p):
            rs, rd, sem = bufs[p]
            off = off_of(c)
            pltpu.async_copy(z_hbm.at[si_v.at[pl.ds(off, K)]], rs, sem)
            pltpu.async_copy(z_hbm.at[di_v.at[pl.ds(off, K)]], rd, sem)

        def wait(p):
            rs, rd, sem = bufs[p]
            pltpu.make_async_copy(
                z_hbm.at[si_v.at[pl.ds(0, K)]], rs, sem).wait()
            pltpu.make_async_copy(
                z_hbm.at[di_v.at[pl.ds(0, K)]], rd, sem).wait()

        def compute(c, p):
            rs, rd, _ = bufs[p]
            off = off_of(c)

            def group_body(g, carry2):
                @plsc.parallel_loop(0, L, step=1, unroll=2)
                def edge_loop(i):
                    e = g * L + i
                    accs = [jnp.zeros((L,), jnp.float32)
                            for _ in range(nacc)]
                    for j in range(D // (2 * L)):
                        s = rs[e, pl.ds(j * 2 * L, 2 * L)]
                        t = rd[e, pl.ds(j * 2 * L, 2 * L)]
                        sa, sb = plsc.unpack(
                            s, format=plsc.PackFormat.INTERLEAVED,
                            preferred_element_type=jnp.float32)
                        ta, tb = plsc.unpack(
                            t, format=plsc.PackFormat.INTERLEAVED,
                            preferred_element_type=jnp.float32)
                        ja = (2 * j) % nacc
                        jb = (2 * j + 1) % nacc
                        accs[ja] = accs[ja] + sa * ta
                        accs[jb] = accs[jb] + sb * tb
                    while len(accs) > 1:
                        accs = [a + b for a, b in zip(accs[::2], accs[1::2])]
                    acc_v[i, pl.ds(0, L)] = accs[0]

                tots = [jnp.zeros((L,), jnp.float32) for _ in range(4)]
                for col in range(L):
                    cvec = jnp.full((L,), col, jnp.int32)
                    tots[col % 4] = tots[col % 4] + plsc.load_gather(
                        acc_v, [lane, cvec])
                dot = (tots[0] + tots[1]) + (tots[2] + tots[3])
                sig = 1.0 / (1.0 + jnp.exp(-dot))
                out_v[pl.ds(off + g * L, L)] = sig
                return carry2

            lax.fori_loop(0, K // L, group_body, 0)

        start(0, 0)

        def pair_body(i, carry):
            for b in range(2):
                c = i * 2 + b

                @pl.when(c + 1 < nchunk)
                def _():
                    start(c + 1, 1 - b)

                @pl.when(c < nchunk)
                def _():
                    wait(b)
                    compute(c, b)
            return carry

        lax.fori_loop(0, npair, pair_body, 0)
        pltpu.sync_copy(out_v, out_hbm.at[pl.ds(base, epw)])

    return recon


_recon = _make(160000, 256, 64)


def kernel(z, edge_index):
    ei = edge_index.astype(jnp.int32)
    return _recon(z.astype(jnp.bfloat16), ei[0], ei[1])


# speedup vs baseline: 10.8532x; 1.0696x over previous
import functools

import jax
import jax.numpy as jnp
from jax import lax
from jax.experimental import pallas as pl
from jax.experimental.pallas import tpu as pltpu
from jax.experimental.pallas import tpu_sc as plsc

NC = 2
NS = 16
L = 16
NW = NC * NS


def _make(E, D, K, nacc=8):
    epw = E // NW
    nchunk = -(-epw // K)
    last_off = epw - K
    npair = -(-nchunk // 2)
    assert E % NW == 0 and K % L == 0 and D % nacc == 0
    assert epw >= K and last_off % 8 == 0 and K <= 128

    mesh = plsc.VectorSubcoreMesh(
        core_axis_name="c", subcore_axis_name="s", num_cores=NC
    )

    @functools.partial(
        pl.kernel,
        mesh=mesh,
        out_type=jax.ShapeDtypeStruct((E,), jnp.float32),
        scratch_types=[
            pltpu.VMEM((epw,), jnp.int32),
            pltpu.VMEM((epw,), jnp.int32),
            pltpu.VMEM((epw,), jnp.float32),
            pltpu.VMEM((K, D), jnp.bfloat16),
            pltpu.VMEM((K, D), jnp.bfloat16),
            pltpu.VMEM((K, D), jnp.bfloat16),
            pltpu.VMEM((K, D), jnp.bfloat16),
            pltpu.VMEM((L, L + 1), jnp.float32),
            pltpu.SemaphoreType.DMA,
            pltpu.SemaphoreType.DMA,
        ],
        compiler_params=pltpu.CompilerParams(
            use_tc_tiling_on_sc=False, needs_layout_passes=False
        ),
    )
    def recon(z_hbm, si_hbm, di_hbm, out_hbm, si_v, di_v, out_v,
              rs0, rd0, rs1, rd1, acc_v, sem0, sem1):
        wid = lax.axis_index("s") * NC + lax.axis_index("c")
        base = wid * epw
        pltpu.sync_copy(si_hbm.at[pl.ds(base, epw)], si_v)
        pltpu.sync_copy(di_hbm.at[pl.ds(base, epw)], di_v)

        lane = lax.iota(jnp.int32, L)
        bufs = ((rs0, rd0, sem0), (rs1, rd1, sem1))

        def off_of(c):
            return jnp.minimum(c * K, last_off)

        def start(c, p):
            rs, rd, sem = bufs[p]
            off = off_of(c)
            pltpu.async_copy(z_hbm.at[si_v.at[pl.ds(off, K)]], rs, sem)
            pltpu.async_copy(z_hbm.at[di_v.at[pl.ds(off, K)]], rd, sem)

        def wait(p):
            rs, rd, sem = bufs[p]
            pltpu.make_async_copy(
                z_hbm.at[si_v.at[pl.ds(0, K)]], rs, sem).wait()
            pltpu.make_async_copy(
                z_hbm.at[di_v.at[pl.ds(0, K)]], rd, sem).wait()

        def compute(c, p):
            rs, rd, _ = bufs[p]
            off = off_of(c)

            def group_body(g, carry2):
                @plsc.parallel_loop(0, L, step=1, unroll=2)
                def edge_loop(i):
                    e = g * L + i
                    accs = [jnp.zeros((L,), jnp.float32)
                            for _ in range(nacc)]
                    for j in range(D // (2 * L)):
                        s = rs[e, pl.ds(j * 2 * L, 2 * L)]
                        t = rd[e, pl.ds(j * 2 * L, 2 * L)]
                        sa, sb = plsc.unpack(
                            s, format=plsc.PackFormat.INTERLEAVED,
                            preferred_element_type=jnp.float32)
                        ta, tb = plsc.unpack(
                            t, format=plsc.PackFormat.INTERLEAVED,
                            preferred_element_type=jnp.float32)
                        ja = (2 * j) % nacc
                        jb = (2 * j + 1) % nacc
                        accs[ja] = accs[ja] + sa * ta
                        accs[jb] = accs[jb] + sb * tb
                    while len(accs) > 1:
                        accs = [a + b for a, b in zip(accs[::2], accs[1::2])]
                    acc_v[i, pl.ds(0, L)] = accs[0]

                tots = [jnp.zeros((L,), jnp.float32) for _ in range(4)]
                for col in range(L):
                    cvec = jnp.full((L,), col, jnp.int32)
                    tots[col % 4] = tots[col % 4] + plsc.load_gather(
                        acc_v, [lane, cvec])
                dot = (tots[0] + tots[1]) + (tots[2] + tots[3])
                sig = 1.0 / (1.0 + jnp.exp(-dot))
                out_v[pl.ds(off + g * L, L)] = sig
                return carry2

            lax.fori_loop(0, K // L, group_body, 0)

        start(0, 0)

        def pair_body(i, carry):
            for b in range(2):
                c = i * 2 + b

                @pl.when(c + 1 < nchunk)
                def _():
                    start(c + 1, 1 - b)

                @pl.when(c < nchunk)
                def _():
                    wait(b)
                    compute(c, b)
            return carry

        lax.fori_loop(0, npair, pair_body, 0)
        pltpu.sync_copy(out_v, out_hbm.at[pl.ds(base, epw)])

    return recon


_recon = _make(160000, 256, 128)


def kernel(z, edge_index):
    ei = edge_index.astype(jnp.int32)
    return _recon(z.astype(jnp.bfloat16), ei[0], ei[1])


# speedup vs baseline: 12.1583x; 1.1202x over previous
import functools

import jax
import jax.numpy as jnp
from jax import lax
from jax.experimental import pallas as pl
from jax.experimental.pallas import tpu as pltpu
from jax.experimental.pallas import tpu_sc as plsc

NC = 2
NS = 16
L = 16
NW = NC * NS


def _make(E, D, K, nacc=8):
    epw = E // NW
    nchunk = -(-epw // K)
    last_off = epw - K
    npair = -(-nchunk // 2)
    assert E % NW == 0 and K % L == 0 and D % nacc == 0
    assert epw >= K and last_off % 8 == 0 and K <= 128

    mesh = plsc.VectorSubcoreMesh(
        core_axis_name="c", subcore_axis_name="s", num_cores=NC
    )

    @functools.partial(
        pl.kernel,
        mesh=mesh,
        out_type=jax.ShapeDtypeStruct((E,), jnp.float32),
        scratch_types=[
            pltpu.VMEM((epw,), jnp.int32),
            pltpu.VMEM((epw,), jnp.int32),
            pltpu.VMEM((epw,), jnp.float32),
            pltpu.VMEM((K, D), jnp.bfloat16),
            pltpu.VMEM((K, D), jnp.bfloat16),
            pltpu.VMEM((K, D), jnp.bfloat16),
            pltpu.VMEM((K, D), jnp.bfloat16),
            pltpu.VMEM((L, L + 1), jnp.float32),
            pltpu.SemaphoreType.DMA,
            pltpu.SemaphoreType.DMA,
        ],
        compiler_params=pltpu.CompilerParams(
            use_tc_tiling_on_sc=False, needs_layout_passes=False
        ),
    )
    def recon(z_hbm, si_hbm, di_hbm, out_hbm, si_v, di_v, out_v,
              rs0, rd0, rs1, rd1, acc_v, sem0, sem1):
        wid = lax.axis_index("s") * NC + lax.axis_index("c")
        base = wid * epw
        pltpu.sync_copy(si_hbm.at[pl.ds(base, epw)], si_v)
        pltpu.sync_copy(di_hbm.at[pl.ds(base, epw)], di_v)

        lane = lax.iota(jnp.int32, L)
        bufs = ((rs0, rd0, sem0), (rs1, rd1, sem1))

        def off_of(c):
            return jnp.minimum(c * K, last_off)

        def start(c, p):
            rs, rd, sem = bufs[p]
            off = off_of(c)
            pltpu.async_copy(z_hbm.at[si_v.at[pl.ds(off, K)]], rs, sem)
            pltpu.async_copy(z_hbm.at[di_v.at[pl.ds(off, K)]], rd, sem)

        def wait(p):
            rs, rd, sem = bufs[p]
            pltpu.make_async_copy(
                z_hbm.at[si_v.at[pl.ds(0, K)]], rs, sem).wait()
            pltpu.make_async_copy(
                z_hbm.at[di_v.at[pl.ds(0, K)]], rd, sem).wait()

        def compute(c, p):
            rs, rd, _ = bufs[p]
            off = off_of(c)

            def group_body(g, carry2):
                @plsc.parallel_loop(0, L, step=1, unroll=2)
                def edge_loop(i):
                    e = g * L + i
                    accs = [jnp.zeros((L,), jnp.float32)
                            for _ in range(nacc)]
                    for j in range(D // (2 * L)):
                        s = rs[e, pl.ds(j * 2 * L, 2 * L)]
                        t = rd[e, pl.ds(j * 2 * L, 2 * L)]
                        pa, pb = plsc.unpack(
                            s * t, format=plsc.PackFormat.INTERLEAVED,
                            preferred_element_type=jnp.float32)
                        ja = (2 * j) % nacc
                        jb = (2 * j + 1) % nacc
                        accs[ja] = accs[ja] + pa
                        accs[jb] = accs[jb] + pb
                    while len(accs) > 1:
                        accs = [a + b for a, b in zip(accs[::2], accs[1::2])]
                    acc_v[i, pl.ds(0, L)] = accs[0]

                tots = [jnp.zeros((L,), jnp.float32) for _ in range(4)]
                for col in range(L):
                    cvec = jnp.full((L,), col, jnp.int32)
                    tots[col % 4] = tots[col % 4] + plsc.load_gather(
                        acc_v, [lane, cvec])
                dot = (tots[0] + tots[1]) + (tots[2] + tots[3])
                sig = 1.0 / (1.0 + jnp.exp(-dot))
                out_v[pl.ds(off + g * L, L)] = sig
                return carry2

            lax.fori_loop(0, K // L, group_body, 0)

        start(0, 0)

        def pair_body(i, carry):
            for b in range(2):
                c = i * 2 + b

                @pl.when(c + 1 < nchunk)
                def _():
                    start(c + 1, 1 - b)

                @pl.when(c < nchunk)
                def _():
                    wait(b)
                    compute(c, b)
            return carry

        lax.fori_loop(0, npair, pair_body, 0)
        pltpu.sync_copy(out_v, out_hbm.at[pl.ds(base, epw)])

    return recon


_recon = _make(160000, 256, 128)


def kernel(z, edge_index):
    ei = edge_index.astype(jnp.int32)
    return _recon(z.astype(jnp.bfloat16), ei[0], ei[1])


# speedup vs baseline: 12.6310x; 1.0389x over previous
import functools

import jax
import jax.numpy as jnp
from jax import lax
from jax.experimental import pallas as pl
from jax.experimental.pallas import tpu as pltpu
from jax.experimental.pallas import tpu_sc as plsc

NC = 2
NS = 16
L = 16
NW = NC * NS


def _make(E, D, K, nacc=8):
    epw = E // NW
    nchunk = -(-epw // K)
    last_off = epw - K
    npair = -(-nchunk // 2)
    assert E % NW == 0 and K % L == 0 and D % nacc == 0
    assert epw >= K and last_off % 8 == 0 and K <= 128

    mesh = plsc.VectorSubcoreMesh(
        core_axis_name="c", subcore_axis_name="s", num_cores=NC
    )

    @functools.partial(
        pl.kernel,
        mesh=mesh,
        out_type=jax.ShapeDtypeStruct((E,), jnp.float32),
        scratch_types=[
            pltpu.VMEM((epw,), jnp.int32),
            pltpu.VMEM((epw,), jnp.int32),
            pltpu.VMEM((epw,), jnp.float32),
            pltpu.VMEM((K, D), jnp.bfloat16),
            pltpu.VMEM((K, D), jnp.bfloat16),
            pltpu.VMEM((K, D), jnp.bfloat16),
            pltpu.VMEM((K, D), jnp.bfloat16),
            pltpu.VMEM((L, L + 1), jnp.float32),
            pltpu.SemaphoreType.DMA,
            pltpu.SemaphoreType.DMA,
        ],
        compiler_params=pltpu.CompilerParams(
            use_tc_tiling_on_sc=False, needs_layout_passes=False
        ),
    )
    def recon(z_hbm, ei_hbm, out_hbm, si_v, di_v, out_v,
              rs0, rd0, rs1, rd1, acc_v, sem0, sem1):
        wid = lax.axis_index("s") * NC + lax.axis_index("c")
        base = wid * epw
        pltpu.sync_copy(ei_hbm.at[0, pl.ds(base, epw)], si_v)
        pltpu.sync_copy(ei_hbm.at[1, pl.ds(base, epw)], di_v)

        lane = lax.iota(jnp.int32, L)
        bufs = ((rs0, rd0, sem0), (rs1, rd1, sem1))

        def off_of(c):
            return jnp.minimum(c * K, last_off)

        def start(c, p):
            rs, rd, sem = bufs[p]
            off = off_of(c)
            pltpu.async_copy(z_hbm.at[si_v.at[pl.ds(off, K)]], rs, sem)
            pltpu.async_copy(z_hbm.at[di_v.at[pl.ds(off, K)]], rd, sem)

        def wait(p):
            rs, rd, sem = bufs[p]
            pltpu.make_async_copy(
                z_hbm.at[si_v.at[pl.ds(0, K)]], rs, sem).wait()
            pltpu.make_async_copy(
                z_hbm.at[di_v.at[pl.ds(0, K)]], rd, sem).wait()

        def compute(c, p):
            rs, rd, _ = bufs[p]
            off = off_of(c)

            def group_body(g, carry2):
                @plsc.parallel_loop(0, L, step=1, unroll=2)
                def edge_loop(i):
                    e = g * L + i
                    accs = [jnp.zeros((L,), jnp.float32)
                            for _ in range(nacc)]
                    for j in range(D // (2 * L)):
                        s = rs[e, pl.ds(j * 2 * L, 2 * L)]
                        t = rd[e, pl.ds(j * 2 * L, 2 * L)]
                        pa, pb = plsc.unpack(
                            s * t, format=plsc.PackFormat.INTERLEAVED,
                            preferred_element_type=jnp.float32)
                        ja = (2 * j) % nacc
                        jb = (2 * j + 1) % nacc
                        accs[ja] = accs[ja] + pa
                        accs[jb] = accs[jb] + pb
                    while len(accs) > 1:
                        accs = [a + b for a, b in zip(accs[::2], accs[1::2])]
                    acc_v[i, pl.ds(0, L)] = accs[0]

                tots = [jnp.zeros((L,), jnp.float32) for _ in range(4)]
                for col in range(L):
                    cvec = jnp.full((L,), col, jnp.int32)
                    tots[col % 4] = tots[col % 4] + plsc.load_gather(
                        acc_v, [lane, cvec])
                dot = (tots[0] + tots[1]) + (tots[2] + tots[3])
                sig = 1.0 / (1.0 + jnp.exp(-dot))
                out_v[pl.ds(off + g * L, L)] = sig
                return carry2

            lax.fori_loop(0, K // L, group_body, 0)

        start(0, 0)

        def pair_body(i, carry):
            for b in range(2):
                c = i * 2 + b

                @pl.when(c + 1 < nchunk)
                def _():
                    start(c + 1, 1 - b)

                @pl.when(c < nchunk)
                def _():
                    wait(b)
                    compute(c, b)
            return carry

        lax.fori_loop(0, npair, pair_body, 0)
        pltpu.sync_copy(out_v, out_hbm.at[pl.ds(base, epw)])

    return recon


_recon = _make(160000, 256, 128)


def kernel(z, edge_index):
    return _recon(z.astype(jnp.bfloat16), edge_index.astype(jnp.int32))
